# baseline (speedup 1.0000x reference)
import functools
import jax
import jax.numpy as jnp
from jax import lax
from jax.experimental import pallas as pl
from jax.experimental.pallas import tpu as pltpu
from jax.experimental.pallas import tpu_sc as plsc

NUM_POINTS = 1000
D = 16
B = 16384
NC = 2
NS = 16
NW = NC * NS
BPW = B // NW
L = 16
GATHER_CHUNK = 128
NGATHER = BPW // GATHER_CHUNK
SUB = GATHER_CHUNK // L

HALF_BANK = 2048
ZERO_BASE = 1000
ZERO_SPAN = 1024
PT_ROWS = 4096

_mesh = plsc.VectorSubcoreMesh(core_axis_name="c", subcore_axis_name="s")


@functools.partial(
    pl.kernel,
    out_type=jax.ShapeDtypeStruct((B, D), jnp.float32),
    mesh=_mesh,
    compiler_params=pltpu.CompilerParams(use_tc_tiling_on_sc=False,
                                         needs_layout_passes=False),
    scratch_types=[
        pltpu.VMEM((BPW,), jnp.float32),
        pltpu.VMEM((NGATHER, GATHER_CHUNK), jnp.int32),
        pltpu.VMEM((NGATHER, GATHER_CHUNK), jnp.int32),
        pltpu.VMEM((BPW, D), jnp.float32),
        [pltpu.SemaphoreType.DMA] * NGATHER,
        [pltpu.SemaphoreType.DMA] * NGATHER,
        [pltpu.SemaphoreType.DMA] * NGATHER,
        pltpu.SemaphoreType.DMA,
        pltpu.VMEM((L,), jnp.int32),
    ],
)
def _sc_lookup(x_hbm, pt_hbm, out_hbm, xv, idx0v, idx1v, rows,
               semx, sem0, sem1, semo, cntv):
    wid = lax.axis_index("s") * NC + lax.axis_index("c")
    base = wid * BPW
    half = jnp.float32(0.5)

    xc = [pltpu.async_copy(x_hbm.at[pl.ds(base + j * GATHER_CHUNK,
                                          GATHER_CHUNK)],
                           xv.at[pl.ds(j * GATHER_CHUNK, GATHER_CHUNK)],
                           semx[j])
          for j in range(NGATHER)]

    g0 = [None] * NGATHER
    cntv[...] = jnp.zeros((L,), jnp.int32)
    for j in range(NGATHER):
        xc[j].wait()
        sacc = jnp.zeros((L,), jnp.int32)
        for ii in range(SUB):
            i = j * SUB + ii
            xl = xv[pl.ds(i * L, L)]
            xs = xl * jnp.float32(NUM_POINTS)
            mi = xs.astype(jnp.int32)
            d0 = xs - mi.astype(jnp.float32)
            d0m = d0 - half
            d1p = (d0 - jnp.float32(1.0)) + half
            ok = mi < NUM_POINTS - 1
            i1 = mi + 1
            ze = ZERO_BASE + ((base + i * L + lax.iota(jnp.int32, 16))
                              & (ZERO_SPAN - 1))
            idxP = jnp.where(d0m < 0.0, mi,
                    jnp.where(d0m == 0.0, mi + HALF_BANK,
                     jnp.where(ok & (d1p > 0.0), i1,
                      jnp.where(ok & (d1p == 0.0), i1 + HALF_BANK, ze))))
            pS = (d1p == 0.0) & ok & (d0m <= 0.0)
            idxS = jnp.where(pS, i1 + HALF_BANK, ze)
            idx0v[j, pl.ds(ii * L, L)] = idxP
            idx1v[j, pl.ds(ii * L, L)] = idxS
            sacc = jnp.maximum(sacc, jnp.where(pS, 1, 0).astype(jnp.int32))
        cntv[...] = jnp.maximum(cntv[...], sacc)
        g0[j] = pltpu.async_copy(
            pt_hbm.at[idx0v.at[j]],
            rows.at[pl.ds(j * GATHER_CHUNK, GATHER_CHUNK)], sem0[j])

    nsec = plsc.all_reduce_population_count(cntv[...] > 0)
    for j in range(NGATHER):
        g0[j].wait()

    @pl.when(nsec[0] > 0)
    def _add_secondaries():
        for j in range(NGATHER):
            pltpu.async_copy(
                pt_hbm.at[idx1v.at[j]],
                rows.at[pl.ds(j * GATHER_CHUNK, GATHER_CHUNK)],
                sem1[j], add=True).wait()

    outs = []
    for j in range(NGATHER):
        outs.append(pltpu.async_copy(
            rows.at[pl.ds(j * GATHER_CHUNK, GATHER_CHUNK)],
            out_hbm.at[pl.ds(base + j * GATHER_CHUNK, GATHER_CHUNK)],
            semo))
    for c in outs:
        c.wait()


def kernel(x, time_embedding):
    pt = jnp.zeros((PT_ROWS, D), jnp.float32)
    pt = pt.at[0:NUM_POINTS].set(time_embedding)
    pt = pt.at[HALF_BANK:HALF_BANK + NUM_POINTS].set(
        time_embedding * jnp.float32(0.5))
    return _sc_lookup(x, pt)


# speedup vs baseline: 1.0489x; 1.0489x over previous
import functools
import jax
import jax.numpy as jnp
from jax import lax
from jax.experimental import pallas as pl
from jax.experimental.pallas import tpu as pltpu
from jax.experimental.pallas import tpu_sc as plsc

NUM_POINTS = 1000
D = 16
B = 16384
NC = 2
NS = 16
NW = NC * NS
BPW = B // NW
L = 16
GATHER_CHUNK = 128
NGATHER = BPW // GATHER_CHUNK
SUB = GATHER_CHUNK // L

HALF_BANK = 2048
ZERO_BASE = 1000
ZERO_SPAN = 1024
PT_ROWS = 4096

_mesh = plsc.VectorSubcoreMesh(core_axis_name="c", subcore_axis_name="s")


@functools.partial(
    pl.kernel,
    out_type=jax.ShapeDtypeStruct((B, D), jnp.float32),
    mesh=_mesh,
    compiler_params=pltpu.CompilerParams(use_tc_tiling_on_sc=False),
    scratch_types=[
        pltpu.VMEM((BPW,), jnp.float32),
        pltpu.VMEM((NGATHER, GATHER_CHUNK), jnp.int32),
        pltpu.VMEM((NGATHER, GATHER_CHUNK), jnp.int32),
        pltpu.VMEM((BPW, D), jnp.float32),
        [pltpu.SemaphoreType.DMA] * NGATHER,
        [pltpu.SemaphoreType.DMA] * NGATHER,
        [pltpu.SemaphoreType.DMA] * NGATHER,
        pltpu.SemaphoreType.DMA,
    ],
)
def _sc_lookup(x_hbm, pt_hbm, out_hbm, xv, idx0v, idx1v, rows,
               semx, sem0, sem1, semo):
    wid = lax.axis_index("s") * NC + lax.axis_index("c")
    base = wid * BPW
    half = jnp.float32(0.5)

    xc = [pltpu.async_copy(x_hbm.at[pl.ds(base + j * GATHER_CHUNK,
                                          GATHER_CHUNK)],
                           xv.at[pl.ds(j * GATHER_CHUNK, GATHER_CHUNK)],
                           semx[j])
          for j in range(NGATHER)]

    g0 = [None] * NGATHER
    for j in range(NGATHER):
        xc[j].wait()
        for ii in range(SUB):
            i = j * SUB + ii
            xl = xv[pl.ds(i * L, L)]
            xs = xl * jnp.float32(NUM_POINTS)
            mi = xs.astype(jnp.int32)
            d0m = (xs - mi.astype(jnp.float32)) - half
            ze = ZERO_BASE + ((base + i * L + lax.iota(jnp.int32, 16))
                              & (ZERO_SPAN - 1))
            idx0 = jnp.where(d0m < 0.0, mi,
                             jnp.where(d0m == 0.0, mi + HALF_BANK, ze))
            idx0v[j, pl.ds(ii * L, L)] = idx0
        g0[j] = pltpu.async_copy(
            pt_hbm.at[idx0v.at[j]],
            rows.at[pl.ds(j * GATHER_CHUNK, GATHER_CHUNK)], sem0[j])

    g1 = [None] * NGATHER
    for j in range(NGATHER):
        for ii in range(SUB):
            i = j * SUB + ii
            xl = xv[pl.ds(i * L, L)]
            xs = xl * jnp.float32(NUM_POINTS)
            mi = xs.astype(jnp.int32)
            d0 = xs - mi.astype(jnp.float32)
            d1p = (d0 - jnp.float32(1.0)) + half
            ok = mi < NUM_POINTS - 1
            i1 = mi + 1
            ze = ZERO_BASE + ((base + i * L + lax.iota(jnp.int32, 16))
                              & (ZERO_SPAN - 1))
            idx1 = jnp.where(ok & (d1p > 0.0), i1,
                             jnp.where(ok & (d1p == 0.0),
                                       i1 + HALF_BANK, ze))
            idx1v[j, pl.ds(ii * L, L)] = idx1
        g0[j].wait()
        g1[j] = pltpu.async_copy(
            pt_hbm.at[idx1v.at[j]],
            rows.at[pl.ds(j * GATHER_CHUNK, GATHER_CHUNK)],
            sem1[j], add=True)

    outs = []
    for j in range(NGATHER):
        g1[j].wait()
        outs.append(pltpu.async_copy(
            rows.at[pl.ds(j * GATHER_CHUNK, GATHER_CHUNK)],
            out_hbm.at[pl.ds(base + j * GATHER_CHUNK, GATHER_CHUNK)],
            semo))
    for c in outs:
        c.wait()


def kernel(x, time_embedding):
    pt = jnp.zeros((PT_ROWS, D), jnp.float32)
    pt = pt.at[0:NUM_POINTS].set(time_embedding)
    pt = pt.at[HALF_BANK:HALF_BANK + NUM_POINTS].set(
        time_embedding * jnp.float32(0.5))
    return _sc_lookup(x, pt)


# speedup vs baseline: 1.0495x; 1.0005x over previous
import functools
import jax
import jax.numpy as jnp
from jax import lax
from jax.experimental import pallas as pl
from jax.experimental.pallas import tpu as pltpu
from jax.experimental.pallas import tpu_sc as plsc

NUM_POINTS = 1000
D = 16
B = 16384
NC = 2
NS = 16
NW = NC * NS
BPW = B // NW
L = 16
GATHER_CHUNK = 128
NGATHER = BPW // GATHER_CHUNK
SUB = GATHER_CHUNK // L

HALF_BANK = 2048
ZERO_BASE = 1000
ZERO_SPAN = 1024
PT_ROWS = 4096

_mesh = plsc.VectorSubcoreMesh(core_axis_name="c", subcore_axis_name="s")


@functools.partial(
    pl.kernel,
    out_type=jax.ShapeDtypeStruct((B, D), jnp.float32),
    mesh=_mesh,
    compiler_params=pltpu.CompilerParams(use_tc_tiling_on_sc=False),
    scratch_types=[
        pltpu.VMEM((BPW,), jnp.float32),
        pltpu.VMEM((NGATHER, GATHER_CHUNK), jnp.int32),
        pltpu.VMEM((NGATHER, GATHER_CHUNK), jnp.int32),
        pltpu.VMEM((BPW, D), jnp.float32),
        [pltpu.SemaphoreType.DMA] * NGATHER,
        [pltpu.SemaphoreType.DMA] * NGATHER,
        [pltpu.SemaphoreType.DMA] * NGATHER,
        pltpu.SemaphoreType.DMA,
    ],
)
def _sc_lookup(x_hbm, pt_hbm, out_hbm, xv, idx0v, idx1v, rows,
               semx, sem0, sem1, semo):
    wid = lax.axis_index("s") * NC + lax.axis_index("c")
    base = wid * BPW
    half = jnp.float32(0.5)

    xc = [pltpu.async_copy(x_hbm.at[pl.ds(base + j * GATHER_CHUNK,
                                          GATHER_CHUNK)],
                           xv.at[pl.ds(j * GATHER_CHUNK, GATHER_CHUNK)],
                           semx[j])
          for j in range(NGATHER)]

    g0 = [None] * NGATHER
    for j in range(NGATHER):
        xc[j].wait()
        for ii in range(SUB):
            i = j * SUB + ii
            xl = xv[pl.ds(i * L, L)]
            xs = xl * jnp.float32(NUM_POINTS)
            mi = xs.astype(jnp.int32)
            d0 = xs - mi.astype(jnp.float32)
            d0m = d0 - half
            d1p = (d0 - jnp.float32(1.0)) + half
            ok = mi < NUM_POINTS - 1
            i1 = mi + 1
            ze = ZERO_BASE + ((base + i * L + lax.iota(jnp.int32, 16))
                              & (ZERO_SPAN - 1))
            idx0 = jnp.where(d0m < 0.0, mi,
                             jnp.where(d0m == 0.0, mi + HALF_BANK, ze))
            idx1 = jnp.where(ok & (d1p > 0.0), i1,
                             jnp.where(ok & (d1p == 0.0),
                                       i1 + HALF_BANK, ze))
            idx0v[j, pl.ds(ii * L, L)] = idx0
            idx1v[j, pl.ds(ii * L, L)] = idx1
        g0[j] = pltpu.async_copy(
            pt_hbm.at[idx0v.at[j]],
            rows.at[pl.ds(j * GATHER_CHUNK, GATHER_CHUNK)], sem0[j])

    g1 = [None] * NGATHER
    for j in range(NGATHER):
        g0[j].wait()
        g1[j] = pltpu.async_copy(
            pt_hbm.at[idx1v.at[j]],
            rows.at[pl.ds(j * GATHER_CHUNK, GATHER_CHUNK)],
            sem1[j], add=True)

    outs = []
    for j in range(NGATHER):
        g1[j].wait()
        outs.append(pltpu.async_copy(
            rows.at[pl.ds(j * GATHER_CHUNK, GATHER_CHUNK)],
            out_hbm.at[pl.ds(base + j * GATHER_CHUNK, GATHER_CHUNK)],
            semo))
    for c in outs:
        c.wait()


def kernel(x, time_embedding):
    pt = jnp.zeros((PT_ROWS, D), jnp.float32)
    pt = pt.at[0:NUM_POINTS].set(time_embedding)
    pt = pt.at[HALF_BANK:HALF_BANK + NUM_POINTS].set(
        time_embedding * jnp.float32(0.5))
    return _sc_lookup(x, pt)
